# baseline (speedup 1.0000x reference)
import jax
import jax.numpy as jnp
from jax.experimental import pallas as pl
from jax.experimental.pallas import tpu as pltpu

_B, _K, _N, _T, _M = 4, 4, 1024, 16, 5
_TM = _T * _M
_BLK = 128


def _spectral_kernel(x_ref, w1_ref, b1_ref, w2_ref, b2_ref, l_ref, out_ref):
    w1c = w1_ref[...].reshape(_TM, _M, _T).sum(axis=1)
    vt = jax.lax.dot_general(w1c, w2_ref[...],
                             (((0,), (1,)), ((), ())),
                             preferred_element_type=jnp.float32)
    beff = jax.lax.dot_general(b1_ref[...], w2_ref[...],
                               (((1,), (1,)), ((), ())),
                               preferred_element_type=jnp.float32)
    beff = beff + b2_ref[...]

    lsum = (l_ref[0] + l_ref[1]) + (l_ref[2] + l_ref[3])
    for b in range(_B):
        s = jnp.dot(lsum, x_ref[b, 0],
                    preferred_element_type=jnp.float32)
        out_ref[b] = jnp.dot(s, vt,
                             preferred_element_type=jnp.float32) + beff


def kernel(x, mul_L, W1, b1, W2, b2):
    return pl.pallas_call(
        _spectral_kernel,
        grid=(_N // _BLK,),
        in_specs=[
            pl.BlockSpec((_B, 1, _N, _T), lambda i: (0, 0, 0, 0)),
            pl.BlockSpec((_TM, _TM), lambda i: (0, 0)),
            pl.BlockSpec((1, _TM), lambda i: (0, 0)),
            pl.BlockSpec((_T, _TM), lambda i: (0, 0)),
            pl.BlockSpec((1, _T), lambda i: (0, 0)),
            pl.BlockSpec((_K, _BLK, _N), lambda i: (0, i, 0)),
        ],
        out_specs=pl.BlockSpec((_B, _BLK, _T), lambda i: (0, i, 0)),
        out_shape=jax.ShapeDtypeStruct((_B, _N, _T), jnp.float32),
        compiler_params=pltpu.CompilerParams(
            dimension_semantics=("parallel",)),
    )(x, W1, b1.reshape(1, _TM), W2, b2.reshape(1, _T), mul_L)


# speedup vs baseline: 1.4974x; 1.4974x over previous
import jax
import jax.numpy as jnp
from jax.experimental import pallas as pl
from jax.experimental.pallas import tpu as pltpu

_B, _K, _N, _T, _M = 4, 4, 1024, 16, 5
_TM = _T * _M
_BT = _B * _T
_CH = 256
_NCH = _N // _CH


def _spectral_kernel(l_hbm, x_ref, w1_ref, b1_ref, w2_ref, b2_ref,
                     out_ref, lbuf, z_ref, sem):
    for k in range(_K):
        for c in range(_NCH):
            pltpu.make_async_copy(
                l_hbm.at[k, pl.ds(c * _CH, _CH), :],
                lbuf.at[k, pl.ds(c * _CH, _CH), :],
                sem.at[k, c]).start()

    w1c = w1_ref[...].reshape(_TM, _M, _T).sum(axis=1)
    vt = jax.lax.dot_general(w1c, w2_ref[...],
                             (((0,), (1,)), ((), ())),
                             preferred_element_type=jnp.float32)
    beff = jax.lax.dot_general(b1_ref[...], w2_ref[...],
                               (((1,), (1,)), ((), ())),
                               preferred_element_type=jnp.float32)
    vb = jnp.tile(beff + b2_ref[...], (1, _B))
    z_ref[...] = jnp.concatenate(
        [jnp.dot(x_ref[b, 0], vt, preferred_element_type=jnp.float32)
         for b in range(_B)], axis=1)

    for c in range(_NCH):
        for k in range(_K):
            pltpu.make_async_copy(
                l_hbm.at[k, pl.ds(c * _CH, _CH), :],
                lbuf.at[k, pl.ds(c * _CH, _CH), :],
                sem.at[k, c]).wait()
        rows = pl.ds(c * _CH, _CH)
        lsum = ((lbuf[0, rows, :] + lbuf[1, rows, :])
                + (lbuf[2, rows, :] + lbuf[3, rows, :]))
        acc = jnp.dot(lsum, z_ref[...],
                      preferred_element_type=jnp.float32) + vb
        for b in range(_B):
            out_ref[b, rows, :] = acc[:, b * _T:(b + 1) * _T]


def kernel(x, mul_L, W1, b1, W2, b2):
    return pl.pallas_call(
        _spectral_kernel,
        in_specs=[
            pl.BlockSpec(memory_space=pltpu.HBM),
            pl.BlockSpec((_B, 1, _N, _T), lambda: (0, 0, 0, 0)),
            pl.BlockSpec((_TM, _TM), lambda: (0, 0)),
            pl.BlockSpec((1, _TM), lambda: (0, 0)),
            pl.BlockSpec((_T, _TM), lambda: (0, 0)),
            pl.BlockSpec((1, _T), lambda: (0, 0)),
        ],
        out_specs=pl.BlockSpec((_B, _N, _T), lambda: (0, 0, 0)),
        out_shape=jax.ShapeDtypeStruct((_B, _N, _T), jnp.float32),
        scratch_shapes=[pltpu.VMEM((_K, _N, _N), jnp.float32),
                        pltpu.VMEM((_N, _BT), jnp.float32),
                        pltpu.SemaphoreType.DMA((_K, _NCH))],
        compiler_params=pltpu.CompilerParams(
            vmem_limit_bytes=50 * 1024 * 1024),
    )(mul_L, x, W1, b1.reshape(1, _TM), W2, b2.reshape(1, _T))


# speedup vs baseline: 1.6159x; 1.0792x over previous
import jax
import jax.numpy as jnp
from jax.experimental import pallas as pl
from jax.experimental.pallas import tpu as pltpu

_B, _K, _N, _T, _M = 4, 4, 1024, 16, 5
_TM = _T * _M
_BT = _B * _T
_CH = 128
_NCH = _N // _CH


def _spectral_kernel(l_hbm, x_ref, w1_ref, b1_ref, w2_ref, b2_ref,
                     out_ref, lbuf, z_ref, sem):
    for c in range(_NCH):
        for k in range(_K):
            pltpu.make_async_copy(
                l_hbm.at[k, pl.ds(c * _CH, _CH), :],
                lbuf.at[k, pl.ds(c * _CH, _CH), :],
                sem.at[k, c]).start()

    w1c = w1_ref[...].reshape(_TM, _M, _T).sum(axis=1)
    vt = jax.lax.dot_general(w1c, w2_ref[...],
                             (((0,), (1,)), ((), ())),
                             preferred_element_type=jnp.float32)
    beff = jax.lax.dot_general(b1_ref[...], w2_ref[...],
                               (((1,), (1,)), ((), ())),
                               preferred_element_type=jnp.float32)
    vb = jnp.tile(beff + b2_ref[...], (1, _B))
    z_ref[...] = jnp.concatenate(
        [jnp.dot(x_ref[b, 0], vt, preferred_element_type=jnp.float32)
         for b in range(_B)], axis=1)

    for c in range(_NCH):
        for k in range(_K):
            pltpu.make_async_copy(
                l_hbm.at[k, pl.ds(c * _CH, _CH), :],
                lbuf.at[k, pl.ds(c * _CH, _CH), :],
                sem.at[k, c]).wait()
        rows = pl.ds(c * _CH, _CH)
        lsum = ((lbuf[0, rows, :] + lbuf[1, rows, :])
                + (lbuf[2, rows, :] + lbuf[3, rows, :]))
        acc = jnp.dot(lsum, z_ref[...],
                      preferred_element_type=jnp.float32) + vb
        for b in range(_B):
            out_ref[b, rows, :] = acc[:, b * _T:(b + 1) * _T]


def kernel(x, mul_L, W1, b1, W2, b2):
    return pl.pallas_call(
        _spectral_kernel,
        in_specs=[
            pl.BlockSpec(memory_space=pltpu.HBM),
            pl.BlockSpec((_B, 1, _N, _T), lambda: (0, 0, 0, 0)),
            pl.BlockSpec((_TM, _TM), lambda: (0, 0)),
            pl.BlockSpec((1, _TM), lambda: (0, 0)),
            pl.BlockSpec((_T, _TM), lambda: (0, 0)),
            pl.BlockSpec((1, _T), lambda: (0, 0)),
        ],
        out_specs=pl.BlockSpec((_B, _N, _T), lambda: (0, 0, 0)),
        out_shape=jax.ShapeDtypeStruct((_B, _N, _T), jnp.float32),
        scratch_shapes=[pltpu.VMEM((_K, _N, _N), jnp.float32),
                        pltpu.VMEM((_N, _BT), jnp.float32),
                        pltpu.SemaphoreType.DMA((_K, _NCH))],
        compiler_params=pltpu.CompilerParams(
            vmem_limit_bytes=50 * 1024 * 1024),
    )(mul_L, x, W1, b1.reshape(1, _TM), W2, b2.reshape(1, _T))


# speedup vs baseline: 1.6296x; 1.0084x over previous
import jax
import jax.numpy as jnp
from jax.experimental import pallas as pl
from jax.experimental.pallas import tpu as pltpu

_B, _K, _N, _T, _M = 4, 4, 1024, 16, 5
_TM = _T * _M
_BT = _B * _T
_CH = 256
_NCH = _N // _CH


def _spectral_kernel(l_hbm, x_ref, w1_ref, b1_ref, w2_ref, b2_ref,
                     out_ref, lbuf, z_ref, sem):
    for c in range(_NCH):
        pltpu.make_async_copy(
            l_hbm.at[:, pl.ds(c * _CH, _CH), :],
            lbuf.at[:, pl.ds(c * _CH, _CH), :],
            sem.at[c]).start()

    w1c = w1_ref[...].reshape(_TM, _M, _T).sum(axis=1)
    vt = jax.lax.dot_general(w1c, w2_ref[...],
                             (((0,), (1,)), ((), ())),
                             preferred_element_type=jnp.float32)
    beff = jax.lax.dot_general(b1_ref[...], w2_ref[...],
                               (((1,), (1,)), ((), ())),
                               preferred_element_type=jnp.float32)
    vb = jnp.tile(beff + b2_ref[...], (1, _B))
    z_ref[...] = jnp.concatenate(
        [jnp.dot(x_ref[b, 0], vt, preferred_element_type=jnp.float32)
         for b in range(_B)], axis=1)

    for c in range(_NCH):
        pltpu.make_async_copy(
            l_hbm.at[:, pl.ds(c * _CH, _CH), :],
            lbuf.at[:, pl.ds(c * _CH, _CH), :],
            sem.at[c]).wait()
        rows = pl.ds(c * _CH, _CH)
        lsum = ((lbuf[0, rows, :] + lbuf[1, rows, :])
                + (lbuf[2, rows, :] + lbuf[3, rows, :]))
        acc = jnp.dot(lsum, z_ref[...],
                      preferred_element_type=jnp.float32) + vb
        for b in range(_B):
            out_ref[b, rows, :] = acc[:, b * _T:(b + 1) * _T]


def kernel(x, mul_L, W1, b1, W2, b2):
    return pl.pallas_call(
        _spectral_kernel,
        in_specs=[
            pl.BlockSpec(memory_space=pltpu.HBM),
            pl.BlockSpec((_B, 1, _N, _T), lambda: (0, 0, 0, 0)),
            pl.BlockSpec((_TM, _TM), lambda: (0, 0)),
            pl.BlockSpec((1, _TM), lambda: (0, 0)),
            pl.BlockSpec((_T, _TM), lambda: (0, 0)),
            pl.BlockSpec((1, _T), lambda: (0, 0)),
        ],
        out_specs=pl.BlockSpec((_B, _N, _T), lambda: (0, 0, 0)),
        out_shape=jax.ShapeDtypeStruct((_B, _N, _T), jnp.float32),
        scratch_shapes=[pltpu.VMEM((_K, _N, _N), jnp.float32),
                        pltpu.VMEM((_N, _BT), jnp.float32),
                        pltpu.SemaphoreType.DMA((_NCH,))],
        compiler_params=pltpu.CompilerParams(
            vmem_limit_bytes=50 * 1024 * 1024),
    )(mul_L, x, W1, b1.reshape(1, _TM), W2, b2.reshape(1, _T))
